# Initial kernel scaffold; baseline (speedup 1.0000x reference)
import jax
import jax.numpy as jnp
from jax.experimental import pallas as pl


def kernel(x, table):
    raise NotImplementedError("write your pallas kernel here")


# speedup vs baseline: 8.1815x; 8.1815x over previous
import functools

import jax
import jax.numpy as jnp
from jax import lax
from jax.experimental import pallas as pl
from jax.experimental.pallas import tpu as pltpu
from jax.experimental.pallas import tpu_sc as plsc

_NC, _NS = 2, 16
_NW = _NC * _NS
_CHUNK = 512


@functools.lru_cache(maxsize=None)
def _make_gather(B, D):
    b_per_w = B // _NW
    num_chunks = b_per_w // _CHUNK
    mesh = plsc.VectorSubcoreMesh(core_axis_name="c", subcore_axis_name="s")

    @functools.partial(
        pl.kernel,
        mesh=mesh,
        out_type=jax.ShapeDtypeStruct((B, D), jnp.float32),
        scratch_types=[
            pltpu.VMEM((_CHUNK,), jnp.int32),
            pltpu.VMEM((_CHUNK, D), jnp.float32),
            pltpu.SemaphoreType.DMA,
        ],
    )
    def gather_kernel(idx_hbm, table_hbm, out_hbm, idx_v, rows_v, sem):
        wid = lax.axis_index("s") * _NC + lax.axis_index("c")
        wbase = wid * b_per_w

        def body(c, carry):
            base = wbase + c * _CHUNK
            pltpu.sync_copy(idx_hbm.at[pl.ds(base, _CHUNK)], idx_v)
            pltpu.async_copy(table_hbm.at[idx_v], rows_v, sem).wait()
            pltpu.sync_copy(rows_v, out_hbm.at[pl.ds(base, _CHUNK)])
            return carry

        lax.fori_loop(0, num_chunks, body, 0)

    return gather_kernel


def kernel(x, table):
    B, L = x.shape
    _, D = table.shape
    idx = x.reshape(-1).astype(jnp.int32)
    out = _make_gather(B * L, D)(idx, table)
    return out.reshape(B, L, D)


# speedup vs baseline: 9.0123x; 1.1015x over previous
import functools

import jax
import jax.numpy as jnp
from jax import lax
from jax.experimental import pallas as pl
from jax.experimental.pallas import tpu as pltpu
from jax.experimental.pallas import tpu_sc as plsc

_NC, _NS = 2, 16
_NW = _NC * _NS
_CHUNK = 320
_NBUF = 2


@functools.lru_cache(maxsize=None)
def _make_gather(B, D):
    b_per_w = B // _NW
    num_chunks = b_per_w // _CHUNK
    num_groups = num_chunks // _NBUF
    mesh = plsc.VectorSubcoreMesh(core_axis_name="c", subcore_axis_name="s")

    @functools.partial(
        pl.kernel,
        mesh=mesh,
        out_type=jax.ShapeDtypeStruct((B, D), jnp.float32),
        scratch_types=[
            pltpu.VMEM((b_per_w,), jnp.int32),
            *[pltpu.VMEM((_CHUNK, D), jnp.float32) for _ in range(_NBUF)],
            *[pltpu.SemaphoreType.DMA for _ in range(2 * _NBUF)],
        ],
    )
    def gather_kernel(idx_hbm, table_hbm, out_hbm, idx_all, *bufs_and_sems):
        rows = bufs_and_sems[:_NBUF]
        gsem = bufs_and_sems[_NBUF:2 * _NBUF]
        ssem = bufs_and_sems[2 * _NBUF:]
        wid = lax.axis_index("s") * _NC + lax.axis_index("c")
        wbase = wid * b_per_w

        def gather(c, b):
            return pltpu.make_async_copy(
                table_hbm.at[idx_all.at[pl.ds(c * _CHUNK, _CHUNK)]],
                rows[b], gsem[b])

        def scatter(c, b):
            return pltpu.make_async_copy(
                rows[b], out_hbm.at[pl.ds(wbase + c * _CHUNK, _CHUNK)],
                ssem[b])

        pltpu.sync_copy(idx_hbm.at[pl.ds(wbase, b_per_w)], idx_all)
        for b in range(_NBUF):
            gather(b, b).start()

        def body(g, carry):
            c0 = g * _NBUF
            for b in range(_NBUF):
                gather(c0 + b, b).wait()
                scatter(c0 + b, b).start()
            for b in range(_NBUF):
                scatter(c0 + b, b).wait()
                gather(c0 + _NBUF + b, b).start()
            return carry

        lax.fori_loop(0, num_groups - 1, body, 0)

        c0 = (num_groups - 1) * _NBUF
        for b in range(_NBUF):
            gather(c0 + b, b).wait()
            scatter(c0 + b, b).start()
        for b in range(_NBUF):
            scatter(c0 + b, b).wait()

    return gather_kernel


def kernel(x, table):
    B, L = x.shape
    _, D = table.shape
    idx = x.reshape(-1).astype(jnp.int32)
    out = _make_gather(B * L, D)(idx, table)
    return out.reshape(B, L, D)
